# Initial kernel scaffold; baseline (speedup 1.0000x reference)
import jax
import jax.numpy as jnp
from jax.experimental import pallas as pl


def kernel(x, E_object_index):
    raise NotImplementedError("write your pallas kernel here")


# speedup vs baseline: 1.3898x; 1.3898x over previous
import jax
import jax.numpy as jnp
from jax import lax
from jax.experimental import pallas as pl
from jax.experimental.pallas import tpu as pltpu
from jax.experimental.pallas import tpu_sc as plsc

OBJ = 1024
ATTR = 8
DIM = 256
BATCH = 4
SEQ = OBJ * ATTR

_info = plsc.get_sparse_core_info()
_NC, _NS, _L = _info.num_cores, _info.num_subcores, _info.num_lanes
_NW = _NC * _NS
_ROWS_W = SEQ // _NW
_CHUNK = 128


def _body(table_hbm, idx_hbm, out_hbm, idx_v, rows_v, sem):
    wid = lax.axis_index("s") * _NC + lax.axis_index("c")
    base = wid * _ROWS_W
    pltpu.sync_copy(idx_hbm.at[wid], idx_v)
    copies = [
        pltpu.async_copy(
            table_hbm.at[idx_v.at[c]],
            rows_v.at[pl.ds(c * _CHUNK, _CHUNK)],
            sem,
        )
        for c in range(_ROWS_W // _CHUNK)
    ]
    for cp in copies:
        cp.wait()
    for b in range(BATCH):
        pltpu.sync_copy(rows_v, out_hbm.at[b, pl.ds(base, _ROWS_W)])


def kernel(x, E_object_index):
    del x
    idx = (jnp.arange(SEQ, dtype=jnp.int32) // ATTR).reshape(
        _NW, _ROWS_W // _CHUNK, _CHUNK
    )
    run = pl.kernel(
        _body,
        out_type=jax.ShapeDtypeStruct((BATCH, SEQ, DIM), jnp.float32),
        mesh=plsc.VectorSubcoreMesh(core_axis_name="c", subcore_axis_name="s"),
        scratch_types=[
            pltpu.VMEM((_ROWS_W // _CHUNK, _CHUNK), jnp.int32),
            pltpu.VMEM((_ROWS_W, DIM), jnp.float32),
            pltpu.SemaphoreType.DMA,
        ],
    )
    return run(E_object_index, idx)
